# baseline (speedup 1.0000x reference)
import jax
import jax.numpy as jnp
from jax.experimental import pallas as pl

_I, _J, _K = 2048, 2048, 16
_BI = 128
_BJ = 128


def _sym_body(a_ref, b_ref, o_ref):
    a = a_ref[...]
    b = b_ref[...]
    o_ref[...] = (a + jnp.swapaxes(b, 0, 1)) * 0.5


def kernel(w):
    W = w.reshape(_I, _J, _K)
    out = pl.pallas_call(
        _sym_body,
        grid=(_I // _BI, _J // _BJ),
        in_specs=[
            pl.BlockSpec((_BI, _BJ, _K), lambda i, j: (i, j, 0)),
            pl.BlockSpec((_BJ, _BI, _K), lambda i, j: (j, i, 0)),
        ],
        out_specs=pl.BlockSpec((_BI, _BJ, _K), lambda i, j: (i, j, 0)),
        out_shape=jax.ShapeDtypeStruct((_I, _J, _K), jnp.float32),
    )(W, W)
    return out.reshape(w.shape)


# speedup vs baseline: 2.6861x; 2.6861x over previous
import jax
import jax.numpy as jnp
from jax.experimental import pallas as pl

_I, _J, _K = 2048, 2048, 16
_BI = 128
_BJ = 128


def _sym_body(a_ref, b_ref, o_ref):
    a = a_ref[...]
    b = b_ref[...]
    t = b.T.reshape(_BI, _K, _BJ).swapaxes(1, 2).reshape(_BI, _BJ * _K)
    o_ref[...] = (a + t) * 0.5


def kernel(w):
    W = w.reshape(_I, _J * _K)
    out = pl.pallas_call(
        _sym_body,
        grid=(_I // _BI, _J // _BJ),
        in_specs=[
            pl.BlockSpec((_BI, _BJ * _K), lambda i, j: (i, j)),
            pl.BlockSpec((_BJ, _BI * _K), lambda i, j: (j, i)),
        ],
        out_specs=pl.BlockSpec((_BI, _BJ * _K), lambda i, j: (i, j)),
        out_shape=jax.ShapeDtypeStruct((_I, _J * _K), jnp.float32),
    )(W, W)
    return out.reshape(w.shape)


# speedup vs baseline: 3.6110x; 1.3443x over previous
import jax
import jax.numpy as jnp
from jax import lax
from jax.experimental import pallas as pl
from jax.experimental.pallas import tpu as pltpu
from jax.experimental.pallas import tpu_sc as plsc

_I, _J, _K = 2048, 2048, 16
_N = _I * _J
_NC, _NS, _L = 2, 16, 16
_NW = _NC * _NS
_E = _J * _K // 128
_IPW = _I // _NW
_S = 128
_NSL = _J // _S


def _sc_transpose_body(w128_hbm, wt_hbm, idx_ref, rows_ref, out_ref, gsems, wsems):
    wid = lax.axis_index("s") * _NC + lax.axis_index("c")
    q0 = wid * (_IPW // 8)
    iota_e = lax.iota(jnp.int32, _L) * _E

    def build_idx(slot, q, j0):
        @pl.loop(0, _S // _L)
        def _(u):
            idx_ref[slot, pl.ds(u * _L, _L)] = iota_e + ((j0 + u * _L) * _E + q)

    def gather(slot):
        return pltpu.make_async_copy(
            w128_hbm.at[idx_ref.at[slot]],
            rows_ref.at[slot],
            gsems.at[slot],
        )

    def write_out(slot, q, j0):
        cps = []
        for r in range(8):
            cps.append(
                pltpu.make_async_copy(
                    out_ref.at[slot, r],
                    wt_hbm.at[pl.ds(((q * 8 + r) * _J + j0) * _K, _S * _K)],
                    wsems.at[slot],
                )
            )
        return cps

    _NIT = (_IPW // 8) * _NSL

    def q_j0(n):
        return q0 + n // _NSL, lax.rem(n, _NSL) * _S

    q_p, j0_p = q_j0(0)
    build_idx(0, q_p, j0_p)
    gather(0).start()

    @pl.loop(1, _NIT + 1)
    def _(n):
        slot = lax.rem(n, 2)
        pslot = 1 - slot

        @pl.when(n < _NIT)
        def _():
            q, j0 = q_j0(n)
            build_idx(slot, q, j0)
            gather(slot).start()

        gather(pslot).wait()

        @pl.when(n >= 3)
        def _():
            qw, j0w = q_j0(n - 3)
            for cp in write_out(pslot, qw, j0w):
                cp.wait()

        @pl.loop(0, _S)
        def _(t):
            for r in range(8):
                out_ref[pslot, r, pl.ds(t * _K, _K)] = rows_ref[
                    pslot, t, pl.ds(r * _L, _L)
                ]

        qp, j0p = q_j0(n - 1)
        for cp in write_out(pslot, qp, j0p):
            cp.start()

    for last in (_NIT - 1, _NIT - 2):
        q, j0 = q_j0(last)
        for cp in write_out(last % 2, q, j0):
            cp.wait()


def _sc_transpose(w128):
    mesh = plsc.VectorSubcoreMesh(core_axis_name="c", subcore_axis_name="s")
    return pl.kernel(
        _sc_transpose_body,
        out_type=jax.ShapeDtypeStruct((_N * _K,), jnp.float32),
        mesh=mesh,
        scratch_types=[
            pltpu.VMEM((2, _S), jnp.int32),
            pltpu.VMEM((2, _S, 128), jnp.float32),
            pltpu.VMEM((2, 8, _S * _K), jnp.float32),
            pltpu.SemaphoreType.DMA((2,)),
            pltpu.SemaphoreType.DMA((2,)),
        ],
    )(w128)


_BR = 32


def _combine_body(a_ref, b_ref, o_ref):
    o_ref[...] = (a_ref[...] + b_ref[...]) * 0.5


def _tc_combine(w2, wt2):
    return pl.pallas_call(
        _combine_body,
        grid=(_I // _BR,),
        in_specs=[
            pl.BlockSpec((_BR, _J * _K), lambda i: (i, 0)),
            pl.BlockSpec((_BR, _J * _K), lambda i: (i, 0)),
        ],
        out_specs=pl.BlockSpec((_BR, _J * _K), lambda i: (i, 0)),
        out_shape=jax.ShapeDtypeStruct((_I, _J * _K), jnp.float32),
    )(w2, wt2)


def kernel(w):
    w128 = w.reshape(_N * _K // 128, 128)
    wt = _sc_transpose(w128)
    out = _tc_combine(w.reshape(_I, _J * _K), wt.reshape(_I, _J * _K))
    return out.reshape(w.shape)


# speedup vs baseline: 6.2521x; 1.7314x over previous
import jax
import jax.numpy as jnp
from jax import lax
from jax.experimental import pallas as pl
from jax.experimental.pallas import tpu as pltpu
from jax.experimental.pallas import tpu_sc as plsc

_I, _J, _K = 2048, 2048, 16
_N = _I * _J
_NC, _NS, _L = 2, 16, 16
_NW = _NC * _NS
_E = _J * _K // 128
_IPW = _I // _NW
_S = 128
_NSL = _J // _S


def _sc_transpose_body(w128_hbm, wt_hbm, idx_ref, rows_ref, out_ref, gsems, wsems):
    wid = lax.axis_index("s") * _NC + lax.axis_index("c")
    q0 = wid * (_IPW // 8)
    iota_e = lax.iota(jnp.int32, _L) * _E

    def build_idx(slot, q, j0):
        @pl.loop(0, _S // _L)
        def _(u):
            idx_ref[slot, pl.ds(u * _L, _L)] = iota_e + ((j0 + u * _L) * _E + q)

    def gather(slot):
        return pltpu.make_async_copy(
            w128_hbm.at[idx_ref.at[slot]],
            rows_ref.at[slot],
            gsems.at[slot],
        )

    def write_out(slot, q, j0):
        cps = []
        for r in range(8):
            row0 = pl.multiple_of(((q * 8 + r) * _J + j0) * _K // 128, 16)
            cps.append(
                pltpu.make_async_copy(
                    out_ref.at[slot, r],
                    wt_hbm.at[pl.ds(row0, _S * _K // 128), :],
                    wsems.at[slot],
                )
            )
        return cps

    _NIT = (_IPW // 8) * _NSL

    def q_j0(n):
        return q0 + n // _NSL, lax.rem(n, _NSL) * _S

    q_p, j0_p = q_j0(0)
    build_idx(0, q_p, j0_p)
    gather(0).start()

    @pl.loop(1, _NIT + 1)
    def _(n):
        slot = lax.rem(n, 2)
        pslot = 1 - slot

        @pl.when(n < _NIT)
        def _():
            q, j0 = q_j0(n)
            build_idx(slot, q, j0)
            gather(slot).start()

        gather(pslot).wait()

        @pl.when(n >= 3)
        def _():
            qw, j0w = q_j0(n - 3)
            for cp in write_out(pslot, qw, j0w):
                cp.wait()

        @pl.loop(0, _S // 8)
        def _(u):
            for v in range(8):
                for r in range(8):
                    out_ref[pslot, r, u, pl.ds(v * _L, _L)] = rows_ref[
                        pslot, u * 8 + v, pl.ds(r * _L, _L)
                    ]

        qp, j0p = q_j0(n - 1)
        for cp in write_out(pslot, qp, j0p):
            cp.start()

    for last in (_NIT - 1, _NIT - 2):
        q, j0 = q_j0(last)
        for cp in write_out(last % 2, q, j0):
            cp.wait()


def _sc_transpose(w128):
    mesh = plsc.VectorSubcoreMesh(core_axis_name="c", subcore_axis_name="s")
    return pl.kernel(
        _sc_transpose_body,
        out_type=jax.ShapeDtypeStruct((_N * _K // 128, 128), jnp.float32),
        mesh=mesh,
        scratch_types=[
            pltpu.VMEM((2, _S), jnp.int32),
            pltpu.VMEM((2, _S, 128), jnp.float32),
            pltpu.VMEM((2, 8, _S * _K // 128, 128), jnp.float32),
            pltpu.SemaphoreType.DMA((2,)),
            pltpu.SemaphoreType.DMA((2,)),
        ],
    )(w128)


_R128 = _N * _K // 128
_BR = 8192


def _combine_body(a_ref, b_ref, o_ref):
    o_ref[...] = (a_ref[...] + b_ref[...]) * 0.5


def _tc_combine(w128, wt128):
    return pl.pallas_call(
        _combine_body,
        grid=(_R128 // _BR,),
        in_specs=[
            pl.BlockSpec((_BR, 128), lambda i: (i, 0)),
            pl.BlockSpec((_BR, 128), lambda i: (i, 0)),
        ],
        out_specs=pl.BlockSpec((_BR, 128), lambda i: (i, 0)),
        out_shape=jax.ShapeDtypeStruct((_R128, 128), jnp.float32),
    )(w128, wt128)


def kernel(w):
    w128 = w.reshape(_R128, 128)
    wt = _sc_transpose(w128)
    out = _tc_combine(w128, wt)
    return out.reshape(w.shape)


# speedup vs baseline: 10.0534x; 1.6080x over previous
import jax
import jax.numpy as jnp
from jax import lax
from jax.experimental import pallas as pl
from jax.experimental.pallas import tpu as pltpu
from jax.experimental.pallas import tpu_sc as plsc

_I, _J, _K = 2048, 2048, 16
_N = _I * _J
_NC, _NS, _L = 2, 16, 16
_NW = _NC * _NS
_E = _J * _K // 128
_IPW = _I // _NW
_S = 128
_NSL = _J // _S


def _sc_transpose_body(w128_hbm, wt_hbm, idx_ref, rows_ref, out_ref, gsems, wsems):
    wid = lax.axis_index("s") * _NC + lax.axis_index("c")
    q0 = wid * (_IPW // 8)
    iota_e = lax.iota(jnp.int32, _L) * _E

    def build_idx(slot, q, j0):
        @pl.loop(0, _S // _L)
        def _(u):
            idx_ref[slot, pl.ds(u * _L, _L)] = iota_e + ((j0 + u * _L) * _E + q)

    def gather(slot):
        return pltpu.make_async_copy(
            w128_hbm.at[idx_ref.at[slot]],
            rows_ref.at[slot],
            gsems.at[slot],
        )

    def write_out(slot, q, j0):
        cps = []
        for r in range(8):
            row0 = pl.multiple_of(((q * 8 + r) * _J + j0) * _K // 128, 16)
            cps.append(
                pltpu.make_async_copy(
                    out_ref.at[slot, r],
                    wt_hbm.at[pl.ds(row0, _S * _K // 128), :],
                    wsems.at[slot],
                )
            )
        return cps

    _NIT = (_IPW // 8) * _NSL

    def q_j0(n):
        return q0 + n // _NSL, lax.rem(n, _NSL) * _S

    q_p, j0_p = q_j0(0)
    build_idx(0, q_p, j0_p)
    gather(0).start()

    @pl.loop(1, _NIT + 1)
    def _(n):
        slot = lax.rem(n, 2)
        pslot = 1 - slot

        @pl.when(n < _NIT)
        def _():
            q, j0 = q_j0(n)
            build_idx(slot, q, j0)
            gather(slot).start()

        gather(pslot).wait()

        @pl.when(n >= 3)
        def _():
            qw, j0w = q_j0(n - 3)
            for cp in write_out(pslot, qw, j0w):
                cp.wait()

        for u in range(_S // 8):
            for v in range(8):
                for r in range(8):
                    out_ref[pslot, r, u, pl.ds(v * _L, _L)] = rows_ref[
                        pslot, u * 8 + v, pl.ds(r * _L, _L)
                    ]

        qp, j0p = q_j0(n - 1)
        for cp in write_out(pslot, qp, j0p):
            cp.start()

    for last in (_NIT - 1, _NIT - 2):
        q, j0 = q_j0(last)
        for cp in write_out(last % 2, q, j0):
            cp.wait()


def _sc_transpose(w128):
    mesh = plsc.VectorSubcoreMesh(core_axis_name="c", subcore_axis_name="s")
    return pl.kernel(
        _sc_transpose_body,
        out_type=jax.ShapeDtypeStruct((_N * _K // 128, 128), jnp.float32),
        mesh=mesh,
        scratch_types=[
            pltpu.VMEM((2, _S), jnp.int32),
            pltpu.VMEM((2, _S, 128), jnp.float32),
            pltpu.VMEM((2, 8, _S * _K // 128, 128), jnp.float32),
            pltpu.SemaphoreType.DMA((2,)),
            pltpu.SemaphoreType.DMA((2,)),
        ],
    )(w128)


_R128 = _N * _K // 128
_BR = 8192


def _combine_body(a_ref, b_ref, o_ref):
    o_ref[...] = (a_ref[...] + b_ref[...]) * 0.5


def _tc_combine(w128, wt128):
    return pl.pallas_call(
        _combine_body,
        grid=(_R128 // _BR,),
        in_specs=[
            pl.BlockSpec((_BR, 128), lambda i: (i, 0)),
            pl.BlockSpec((_BR, 128), lambda i: (i, 0)),
        ],
        out_specs=pl.BlockSpec((_BR, 128), lambda i: (i, 0)),
        out_shape=jax.ShapeDtypeStruct((_R128, 128), jnp.float32),
    )(w128, wt128)


def kernel(w):
    w128 = w.reshape(_R128, 128)
    wt = _sc_transpose(w128)
    out = _tc_combine(w128, wt)
    return out.reshape(w.shape)


# speedup vs baseline: 10.0629x; 1.0009x over previous
import jax
import jax.numpy as jnp
from jax import lax
from jax.experimental import pallas as pl
from jax.experimental.pallas import tpu as pltpu
from jax.experimental.pallas import tpu_sc as plsc

_I, _J, _K = 2048, 2048, 16
_N = _I * _J
_NC, _NS, _L = 2, 16, 16
_NW = _NC * _NS
_E = _J * _K // 128
_S = 128
_NSL = _J // _S
_NSLICE = 4
_IPWS = _I // _NW // _NSLICE
_R128 = _N * _K // 128
_RSL = _R128 // _NSLICE


def _sc_transpose_body(s, w128_hbm, wt_hbm, idx_ref, rows_ref, out_ref, gsems, wsems):
    wid = lax.axis_index("s") * _NC + lax.axis_index("c")
    q0 = s * (_I // 8 // _NSLICE) + wid * (_IPWS // 8)
    iota_e = lax.iota(jnp.int32, _L) * _E

    def build_idx(slot, q, j0):
        @pl.loop(0, _S // _L)
        def _(u):
            idx_ref[slot, pl.ds(u * _L, _L)] = iota_e + ((j0 + u * _L) * _E + q)

    def gather(slot):
        return pltpu.make_async_copy(
            w128_hbm.at[idx_ref.at[slot]],
            rows_ref.at[slot],
            gsems.at[slot],
        )

    def write_out(slot, q, j0):
        cps = []
        for r in range(8):
            gr = ((q * 8 + r - s * (_I // _NSLICE)) * _J + j0) * _K // 128
            row0 = pl.multiple_of(gr, 16)
            cps.append(
                pltpu.make_async_copy(
                    out_ref.at[slot, r],
                    wt_hbm.at[pl.ds(row0, _S * _K // 128), :],
                    wsems.at[slot],
                )
            )
        return cps

    _NIT = (_IPWS // 8) * _NSL

    def q_j0(n):
        return q0 + n // _NSL, lax.rem(n, _NSL) * _S

    q_p, j0_p = q_j0(0)
    build_idx(0, q_p, j0_p)
    gather(0).start()

    @pl.loop(1, _NIT + 1)
    def _(n):
        slot = lax.rem(n, 2)
        pslot = 1 - slot

        @pl.when(n < _NIT)
        def _():
            q, j0 = q_j0(n)
            build_idx(slot, q, j0)
            gather(slot).start()

        gather(pslot).wait()

        @pl.when(n >= 3)
        def _():
            qw, j0w = q_j0(n - 3)
            for cp in write_out(pslot, qw, j0w):
                cp.wait()

        for u in range(_S // 8):
            for v in range(8):
                for r in range(8):
                    out_ref[pslot, r, u, pl.ds(v * _L, _L)] = rows_ref[
                        pslot, u * 8 + v, pl.ds(r * _L, _L)
                    ]

        qp, j0p = q_j0(n - 1)
        for cp in write_out(pslot, qp, j0p):
            cp.start()

    for last in (_NIT - 1, _NIT - 2):
        q, j0 = q_j0(last)
        for cp in write_out(last % 2, q, j0):
            cp.wait()


def _sc_transpose_slice(s, w128):
    mesh = plsc.VectorSubcoreMesh(core_axis_name="c", subcore_axis_name="s")

    def body(*refs):
        _sc_transpose_body(s, *refs)

    return pl.kernel(
        body,
        out_type=jax.ShapeDtypeStruct((_RSL, 128), jnp.float32),
        mesh=mesh,
        scratch_types=[
            pltpu.VMEM((2, _S), jnp.int32),
            pltpu.VMEM((2, _S, 128), jnp.float32),
            pltpu.VMEM((2, 8, _S * _K // 128, 128), jnp.float32),
            pltpu.SemaphoreType.DMA((2,)),
            pltpu.SemaphoreType.DMA((2,)),
        ],
        name=f"sc_transpose_s{s}",
    )(w128)


_BR = 8192
_NBLK = _RSL // _BR


def _combine_body(a_ref, b_ref, o_ref):
    o_ref[...] = (a_ref[...] + b_ref[...]) * 0.5


def _combine_body_carry(carry_hbm, a_ref, b_ref, o_ref):
    del carry_hbm
    o_ref[...] = (a_ref[...] + b_ref[...]) * 0.5


def _tc_combine_slice(s, w128, wt_s, carry):
    imap = lambda i: (s * _NBLK + i, 0)
    wt_map = lambda i: (i, 0)
    if carry is None:
        return pl.pallas_call(
            _combine_body,
            grid=(_NBLK,),
            in_specs=[
                pl.BlockSpec((_BR, 128), imap),
                pl.BlockSpec((_BR, 128), wt_map),
            ],
            out_specs=pl.BlockSpec((_BR, 128), imap),
            out_shape=jax.ShapeDtypeStruct((_R128, 128), jnp.float32),
        )(w128, wt_s)
    return pl.pallas_call(
        _combine_body_carry,
        grid=(_NBLK,),
        in_specs=[
            pl.BlockSpec(memory_space=pl.ANY),
            pl.BlockSpec((_BR, 128), imap),
            pl.BlockSpec((_BR, 128), wt_map),
        ],
        out_specs=pl.BlockSpec((_BR, 128), imap),
        out_shape=jax.ShapeDtypeStruct((_R128, 128), jnp.float32),
        input_output_aliases={0: 0},
    )(carry, w128, wt_s)


def kernel(w):
    w128 = w.reshape(_R128, 128)
    wts = [_sc_transpose_slice(s, w128) for s in range(_NSLICE)]
    out = None
    for s in range(_NSLICE):
        out = _tc_combine_slice(s, w128, wts[s], out)
    return out.reshape(w.shape)
